# Initial kernel scaffold; baseline (speedup 1.0000x reference)
import jax
import jax.numpy as jnp
from jax.experimental import pallas as pl


def kernel(feat_index, feat_value, emb_table, weight, bias):
    raise NotImplementedError("write your pallas kernel here")


# speedup vs baseline: 1.1754x; 1.1754x over previous
import jax
import jax.numpy as jnp
from jax import lax
from jax.experimental import pallas as pl
from jax.experimental.pallas import tpu as pltpu
from jax.experimental.pallas import tpu_sc as plsc

B = 16384
F = 26
E = 16
NC, NS = 2, 16
NW = NC * NS
SPW = B // NW
RPW = SPW * F
G = 128
GPW = RPW // G
NCHUNK = 4
CS = SPW // NCHUNK
GPC = GPW // NCHUNK
RPC = CS * F


def _sc_body(idx_hbm, val_hbm, table_hbm, w_hbm, b_hbm, out_hbm,
             idx_v, val_v, rows_v, out_v, w_v, b_v, sem):
    wid = lax.axis_index("s") * NC + lax.axis_index("c")

    pltpu.sync_copy(idx_hbm.at[pl.ds(wid * GPW, GPW)], idx_v)
    pltpu.sync_copy(val_hbm.at[pl.ds(wid * RPW, RPW)], val_v)
    pltpu.sync_copy(w_hbm, w_v)
    pltpu.sync_copy(b_hbm, b_v)

    wv = w_v[...] * jnp.float32(1.0 / F)
    bs = b_v[...][0]
    onehot = [lax.iota(jnp.int32, E) == k for k in range(E)]

    def chunk_body(c, carry):
        copies = []
        for j in range(GPC):
            cp = pltpu.make_async_copy(
                table_hbm.at[idx_v.at[c * GPC + j]],
                rows_v.at[pl.ds(j * G, G)],
                sem,
            )
            cp.start()
            copies.append(cp)
        for cp in copies:
            cp.wait()

        def group_body(g, carry2):
            svec = jnp.zeros((E,), jnp.float32)
            for k in range(E):
                n0 = (g * E + k) * F
                m0 = c * RPC + n0
                va = val_v[pl.ds(m0, E)]
                vb = val_v[pl.ds(m0 + 10, E)]
                acc = rows_v[n0, :] * va[0]
                for f in range(1, E):
                    acc = acc + rows_v[n0 + f, :] * va[f]
                for f in range(E, F):
                    acc = acc + rows_v[n0 + f, :] * vb[f - 10]
                t = acc * wv
                for d in (8, 4, 2, 1):
                    perm = lax.iota(jnp.int32, E) ^ d
                    t = t + t.at[perm].get(mode="promise_in_bounds")
                svec = jnp.where(onehot[k], t, svec)
            out_v[pl.ds(c * CS + g * E, E)] = svec
            return carry2

        lax.fori_loop(0, CS // E, group_body, 0)
        return carry

    lax.fori_loop(0, NCHUNK, chunk_body, 0)

    for i in range(SPW // E):
        x = out_v[pl.ds(i * E, E)] + bs
        out_v[pl.ds(i * E, E)] = 1.0 / (1.0 + jnp.exp(-x))

    pltpu.sync_copy(out_v, out_hbm.at[pl.ds(wid * SPW, SPW)])


@jax.jit
def _lr_sc(idx2, valf, table, w16, b16):
    run = pl.kernel(
        _sc_body,
        out_type=jax.ShapeDtypeStruct((B,), jnp.float32),
        mesh=plsc.VectorSubcoreMesh(core_axis_name="c", subcore_axis_name="s"),
        scratch_types=[
            pltpu.VMEM((GPW, G), jnp.int32),
            pltpu.VMEM((RPW,), jnp.float32),
            pltpu.VMEM((RPC, E), jnp.float32),
            pltpu.VMEM((SPW,), jnp.float32),
            pltpu.VMEM((E,), jnp.float32),
            pltpu.VMEM((E,), jnp.float32),
            pltpu.SemaphoreType.DMA,
        ],
        compiler_params=pltpu.CompilerParams(use_tc_tiling_on_sc=False),
    )
    return run(idx2, valf, table, w16, b16)


def kernel(feat_index, feat_value, emb_table, weight, bias):
    idx2 = feat_index.reshape(B * F // G, G)
    valf = feat_value.reshape(B * F)
    w16 = weight.reshape(E)
    b16 = jnp.broadcast_to(bias, (E,))
    out = _lr_sc(idx2, valf, emb_table, w16, b16)
    return out.reshape(B, 1)
